# baseline (speedup 1.0000x reference)
import jax
import jax.numpy as jnp
from jax import lax
from jax.experimental import pallas as pl
from jax.experimental.pallas import tpu as pltpu
from jax.experimental.pallas import tpu_sc as plsc

SPATIAL = 64
C = 32
NV = 2 * SPATIAL ** 3
NC = 2
NS = 16
LANES = 16

W = 48128
NWIN = 11
NPASS = 6
TAIL_W = NV - (NWIN - 1) * W
TRASH = W
SH_ROWS = W + 8

STRIPE = W // NS
TAIL_STRIPE = TAIL_W // NS
CHUNK = 128
NBUF = 4
NZFULL = STRIPE // CHUNK
ZREM = STRIPE - NZFULL * CHUNK
UNROLL = 4
SEG = 3136

N_POINTS = 200000
NSL = -(-N_POINTS // (NS * LANES * UNROLL)) * (LANES * UNROLL)
N_PAD = NSL * NS
NSEG = NSL // SEG


def _sc_body(b_hbm, x_hbm, y_hbm, z_hbm, feats_hbm, out_hbm,
             flat_v, selp_v, pidc_vs, dstc_vs, feat_vs, shared, sems):
    c = lax.axis_index("c")
    s = lax.axis_index("s")
    sbase = s * NSL

    for d, src in enumerate((b_hbm, x_hbm, y_hbm, z_hbm)):
        for t in range(NSL // SEG):
            pltpu.sync_copy(src.at[pl.ds(sbase + t * SEG, SEG)], selp_v)

            def fb(i, carry):
                sl = pl.ds(t * SEG + i * LANES, LANES)
                cv = selp_v[pl.ds(i * LANES, LANES)]
                if d == 0:
                    flat_v[sl] = cv
                else:
                    flat_v[sl] = flat_v[sl] * SPATIAL + cv
                return carry

            lax.fori_loop(0, SEG // LANES, fb, 0)

    zf = jnp.zeros((LANES,), jnp.float32)

    def build_idx(t, j, nsel, lo):
        cb0 = j * CHUNK
        for k in range(CHUNK // LANES):
            off2 = cb0 + k * LANES
            lane = off2 + lax.iota(jnp.int32, LANES)
            mm = lane < nsel
            pv = jnp.where(mm, selp_v[pl.ds(off2, LANES)], 0)
            fv = plsc.load_gather(flat_v, [pv])
            pidc_vs[t][pl.ds(k * LANES, LANES)] = jnp.where(mm, pv + sbase, 0)
            dstc_vs[t][pl.ds(k * LANES, LANES)] = jnp.where(mm, fv - lo,
                                                            TRASH)

    def fire(t):
        pltpu.async_copy(feats_hbm.at[pidc_vs[t]], feat_vs[t], sems[t])

    def wait(t):
        pltpu.make_async_copy(feats_hbm.at[pidc_vs[t]], feat_vs[t],
                              sems[t]).wait()

    def pass_body(p, carry):
        wid = p * NC + c
        lo = wid * W
        in_range = lo < NV

        @pl.when(in_range)
        def _pass_zero():
            def zb(i, carry2):
                feat_vs[0][i, pl.ds(0, LANES)] = zf
                feat_vs[0][i, pl.ds(LANES, LANES)] = zf
                return carry2

            lax.fori_loop(0, CHUNK, zb, 0)
            for t in range(NZFULL):
                pltpu.sync_copy(
                    feat_vs[0],
                    shared.at[pl.ds(s * STRIPE + t * CHUNK, CHUNK)])
            pltpu.sync_copy(
                feat_vs[0].at[pl.ds(0, ZREM)],
                shared.at[pl.ds(s * STRIPE + NZFULL * CHUNK, ZREM)])

        plsc.subcore_barrier()

        @pl.when(in_range)
        def _route_work():
            def seg_body(g, carry3):
                gbase = g * SEG

                def cb(i, cur):
                    base = gbase + i * (LANES * UNROLL)
                    vs, ms = [], []
                    for u in range(UNROLL):
                        v = flat_v[pl.ds(base + u * LANES, LANES)]
                        ms.append((v >= lo) & (v < lo + W))
                        vs.append(v)
                    inc = cur
                    for u in range(UNROLL):
                        ones = jnp.where(ms[u], 1, 0).astype(jnp.int32)
                        pos = inc + plsc.cumsum(ones) - 1
                        lid = (base + u * LANES) + lax.iota(jnp.int32, LANES)
                        plsc.store_scatter(selp_v, [pos], lid, mask=ms[u])
                        inc = inc + plsc.all_reduce_population_count(ms[u])
                    return inc

                curf = lax.fori_loop(0, SEG // (LANES * UNROLL), cb,
                                     jnp.zeros((LANES,), jnp.int32))
                nsel = jnp.max(curf)
                nch = (nsel + (CHUNK - 1)) // CHUNK

                for t in range(NBUF):
                    @pl.when(t < nch)
                    def _prime(t=t):
                        build_idx(t, jnp.int32(t), nsel, lo)
                        fire(t)

                def hb(jj, carry4):
                    for t in range(NBUF):
                        j = NBUF * jj + t

                        @pl.when(j < nch)
                        def _step(t=t, j=j):
                            wait(t)
                            pltpu.sync_copy(feat_vs[t],
                                            shared.at[dstc_vs[t]], add=True)

                            @pl.when(j + NBUF < nch)
                            def _refill(t=t, j=j):
                                build_idx(t, j + NBUF, nsel, lo)
                                fire(t)

                    return carry4

                lax.fori_loop(0, (nch + NBUF - 1) // NBUF, hb, 0)
                return carry3

            lax.fori_loop(0, NSEG, seg_body, 0)

        plsc.subcore_barrier()

        full = lo + W <= NV

        @pl.when(in_range & full)
        def _drain_full():
            pltpu.sync_copy(shared.at[pl.ds(s * STRIPE, STRIPE)],
                            out_hbm.at[pl.ds(lo + s * STRIPE, STRIPE)])

        @pl.when(in_range & jnp.logical_not(full))
        def _drain_tail():
            pltpu.sync_copy(
                shared.at[pl.ds(s * TAIL_STRIPE, TAIL_STRIPE)],
                out_hbm.at[pl.ds(lo + s * TAIL_STRIPE, TAIL_STRIPE)])

        return carry

    lax.fori_loop(0, NPASS, pass_body, 0)


def kernel(coords, features, batch_idx, batch_size):
    n = coords.shape[0]
    shift = jnp.asarray(batch_size, jnp.int32) - 2
    pad = N_PAD - n
    b_a = jnp.pad(batch_idx.astype(jnp.int32), (0, pad), constant_values=-1)
    x_a = jnp.pad(coords[:, 0].astype(jnp.int32), (0, pad),
                  constant_values=-1)
    y_a = jnp.pad(coords[:, 1].astype(jnp.int32), (0, pad),
                  constant_values=-1)
    z_a = jnp.pad(coords[:, 2].astype(jnp.int32) + shift, (0, pad),
                  constant_values=-1)
    feats = features.astype(jnp.float32)

    mesh = plsc.VectorSubcoreMesh(core_axis_name="c", subcore_axis_name="s",
                                  num_cores=NC, num_subcores=NS)
    run = pl.kernel(
        _sc_body,
        out_type=jax.ShapeDtypeStruct((NV, C), jnp.float32),
        mesh=mesh,
        scratch_types=[
            pltpu.VMEM((NSL,), jnp.int32),
            pltpu.VMEM((SEG,), jnp.int32),
            [pltpu.VMEM((CHUNK,), jnp.int32) for _ in range(NBUF)],
            [pltpu.VMEM((CHUNK,), jnp.int32) for _ in range(NBUF)],
            [pltpu.VMEM((CHUNK, C), jnp.float32) for _ in range(NBUF)],
            pltpu.VMEM_SHARED((SH_ROWS, C), jnp.float32),
            [pltpu.SemaphoreType.DMA for _ in range(NBUF)],
        ],
        compiler_params=pltpu.CompilerParams(needs_layout_passes=False,
                                             use_tc_tiling_on_sc=False),
    )
    return run(b_a, x_a, y_a, z_a, feats)


# speedup vs baseline: 1.3608x; 1.3608x over previous
import jax
import jax.numpy as jnp
from jax import lax
from jax.experimental import pallas as pl
from jax.experimental.pallas import tpu as pltpu
from jax.experimental.pallas import tpu_sc as plsc

SPATIAL = 64
C = 32
NV = 2 * SPATIAL ** 3
NC = 2
NS = 16
LANES = 16

W = 53248
NWIN = 10
NPASS = 5
TAIL_W = NV - (NWIN - 1) * W
TRASH = W
SH_ROWS = W + 128

STRIPE = W // NS
TAIL_STRIPE = TAIL_W // NS
BLK = 128
SEG = 3136

N_POINTS = 200000
NSL = -(-N_POINTS // (NS * BLK)) * BLK
N_PAD = NSL * NS
NBLK = NSL // BLK


def _sc_body(b_hbm, x_hbm, y_hbm, z_hbm, feats_hbm, out_hbm,
             flat_v, stg_v, dstc0_v, dstc1_v, fbuf0_v, fbuf1_v,
             shared, sem0, sem1):
    c = lax.axis_index("c")
    s = lax.axis_index("s")
    sbase = s * NSL

    for d, src in enumerate((b_hbm, x_hbm, y_hbm, z_hbm)):
        for t in range(NSL // SEG):
            pltpu.sync_copy(src.at[pl.ds(sbase + t * SEG, SEG)], stg_v)

            def fb(i, carry):
                sl = pl.ds(t * SEG + i * LANES, LANES)
                cv = stg_v[pl.ds(i * LANES, LANES)]
                if d == 0:
                    flat_v[sl] = cv
                else:
                    flat_v[sl] = flat_v[sl] * SPATIAL + cv
                return carry

            lax.fori_loop(0, SEG // LANES, fb, 0)

    zf = jnp.zeros((LANES,), jnp.float32)

    def build_dst(dstc, blk, lo):
        base = blk * BLK
        for k in range(BLK // LANES):
            v = flat_v[pl.ds(base + k * LANES, LANES)]
            m = (v >= lo) & (v < lo + W)
            trash = (TRASH + k * LANES) + lax.iota(jnp.int32, LANES)
            dstc[pl.ds(k * LANES, LANES)] = jnp.where(m, v - lo, trash)

    def stream(fbuf, blk, sem):
        pltpu.async_copy(
            feats_hbm.at[pl.ds(sbase + blk * BLK, BLK)], fbuf, sem)

    def swait(fbuf, blk, sem):
        pltpu.make_async_copy(
            feats_hbm.at[pl.ds(sbase + blk * BLK, BLK)], fbuf, sem).wait()

    def pass_body(p, carry):
        wid = p * NC + c
        lo = wid * W
        in_range = lo < NV

        @pl.when(in_range)
        def _pass_zero():
            def zb(i, carry2):
                fbuf0_v[i, pl.ds(0, LANES)] = zf
                fbuf0_v[i, pl.ds(LANES, LANES)] = zf
                return carry2

            lax.fori_loop(0, BLK, zb, 0)
            for t in range(STRIPE // BLK):
                pltpu.sync_copy(
                    fbuf0_v, shared.at[pl.ds(s * STRIPE + t * BLK, BLK)])

        plsc.subcore_barrier()

        @pl.when(in_range)
        def _route_work():
            stream(fbuf0_v, jnp.int32(0), sem0)

            def hb(bb, carry3):
                b0 = 2 * bb
                b1 = 2 * bb + 1
                stream(fbuf1_v, b1, sem1)
                swait(fbuf0_v, b0, sem0)
                build_dst(dstc0_v, b0, lo)
                pltpu.sync_copy(fbuf0_v, shared.at[dstc0_v], add=True)

                @pl.when(b0 + 2 < NBLK)
                def _refill0():
                    stream(fbuf0_v, b0 + 2, sem0)

                swait(fbuf1_v, b1, sem1)
                build_dst(dstc1_v, b1, lo)
                pltpu.sync_copy(fbuf1_v, shared.at[dstc1_v], add=True)
                return carry3

            lax.fori_loop(0, NBLK // 2, hb, 0)

        plsc.subcore_barrier()

        full = lo + W <= NV

        @pl.when(in_range & full)
        def _drain_full():
            pltpu.sync_copy(shared.at[pl.ds(s * STRIPE, STRIPE)],
                            out_hbm.at[pl.ds(lo + s * STRIPE, STRIPE)])

        @pl.when(in_range & jnp.logical_not(full))
        def _drain_tail():
            pltpu.sync_copy(
                shared.at[pl.ds(s * TAIL_STRIPE, TAIL_STRIPE)],
                out_hbm.at[pl.ds(lo + s * TAIL_STRIPE, TAIL_STRIPE)])

        return carry

    lax.fori_loop(0, NPASS, pass_body, 0)


def kernel(coords, features, batch_idx, batch_size):
    n = coords.shape[0]
    shift = jnp.asarray(batch_size, jnp.int32) - 2
    pad = N_PAD - n
    b_a = jnp.pad(batch_idx.astype(jnp.int32), (0, pad), constant_values=-1)
    x_a = jnp.pad(coords[:, 0].astype(jnp.int32), (0, pad),
                  constant_values=-1)
    y_a = jnp.pad(coords[:, 1].astype(jnp.int32), (0, pad),
                  constant_values=-1)
    z_a = jnp.pad(coords[:, 2].astype(jnp.int32) + shift, (0, pad),
                  constant_values=-1)
    feats = jnp.pad(features.astype(jnp.float32), ((0, pad), (0, 0)))

    mesh = plsc.VectorSubcoreMesh(core_axis_name="c", subcore_axis_name="s",
                                  num_cores=NC, num_subcores=NS)
    run = pl.kernel(
        _sc_body,
        out_type=jax.ShapeDtypeStruct((NV, C), jnp.float32),
        mesh=mesh,
        scratch_types=[
            pltpu.VMEM((NSL,), jnp.int32),
            pltpu.VMEM((SEG,), jnp.int32),
            pltpu.VMEM((BLK,), jnp.int32),
            pltpu.VMEM((BLK,), jnp.int32),
            pltpu.VMEM((BLK, C), jnp.float32),
            pltpu.VMEM((BLK, C), jnp.float32),
            pltpu.VMEM_SHARED((SH_ROWS, C), jnp.float32),
            pltpu.SemaphoreType.DMA,
            pltpu.SemaphoreType.DMA,
        ],
        compiler_params=pltpu.CompilerParams(needs_layout_passes=False,
                                             use_tc_tiling_on_sc=False),
    )
    return run(b_a, x_a, y_a, z_a, feats)
